# baseline (speedup 1.0000x reference)
import jax
import jax.numpy as jnp
from jax import lax
from jax.experimental import pallas as pl
from jax.experimental.pallas import tpu as pltpu
from jax.experimental.pallas import tpu_sc as plsc

N = 10000
E = 320000
F = 128
G = 8
C = 1000

NC, NS = 2, 16
NW = NC * NS
EPW = E // NW
CH = 80
FH = F // NC
EPT = E // NS
NCHT = EPT // CH
NCHUNK = EPW // CH
RPW = N // NS
NFULL = RPW // CH
TAIL = RPW - NFULL * CH
DEGW = 16

BLK = 1000
NBLK = N // BLK

_MESH = plsc.VectorSubcoreMesh(
    core_axis_name="c", subcore_axis_name="s", num_cores=NC, num_subcores=NS)

_SC_PARAMS = pltpu.CompilerParams(use_tc_tiling_on_sc=False)


def _deg_body(dst_hbm, out_hbm, dst_v, ones_v, z_v, deg_sh):
    c = lax.axis_index("c")
    s = lax.axis_index("s")
    w = s * NC + c

    @pl.loop(0, CH)
    def _(i):
        ones_v[i] = jnp.ones((DEGW,), jnp.float32)

    @pl.loop(0, RPW)
    def _(i):
        z_v[i] = jnp.zeros((DEGW,), jnp.float32)

    pltpu.sync_copy(z_v, deg_sh.at[pl.ds(s * RPW, RPW)])
    plsc.subcore_barrier()

    pltpu.sync_copy(dst_hbm.at[pl.ds(w * NCHUNK, NCHUNK)], dst_v)

    @pl.loop(0, NCHUNK)
    def _(j):
        pltpu.sync_copy(ones_v, deg_sh.at[dst_v.at[j]], add=True)

    plsc.subcore_barrier()
    pltpu.sync_copy(deg_sh.at[pl.ds(s * RPW, RPW)], z_v)
    pltpu.sync_copy(z_v, out_hbm.at[c, pl.ds(s * RPW, RPW)])


_deg_call = pl.kernel(
    _deg_body,
    out_type=jax.ShapeDtypeStruct((NC, N, DEGW), jnp.float32),
    mesh=_MESH,
    scratch_types=[
        pltpu.VMEM((NCHUNK, CH), jnp.int32),
        pltpu.VMEM((CH, DEGW), jnp.float32),
        pltpu.VMEM((RPW, DEGW), jnp.float32),
        pltpu.VMEM_SHARED((N, DEGW), jnp.float32),
    ],
    compiler_params=_SC_PARAMS,
)


def _edge_body(hp_hbm, src_hbm, dst_hbm, out_hbm, src_v, dst_v, rows, gsems,
               ssems, acc_sh):
    c = lax.axis_index("c")
    s = lax.axis_index("s")
    w = s * NC + c

    pltpu.sync_copy(src_hbm.at[pl.ds(w * NCHUNK, NCHUNK)], src_v)
    pltpu.sync_copy(dst_hbm.at[pl.ds(w * NCHUNK, NCHUNK)], dst_v)

    @pl.loop(0, CH)
    def _(i):
        for jj in range(F // 16):
            rows[0][i, pl.ds(jj * 16, 16)] = jnp.zeros((16,), jnp.float32)

    for k in range(NFULL):
        pltpu.async_copy(rows[0], acc_sh.at[pl.ds(s * RPW + k * CH, CH)],
                         ssems[0])
    pltpu.async_copy(rows[0].at[pl.ds(0, TAIL)],
                     acc_sh.at[pl.ds(s * RPW + NFULL * CH, TAIL)], ssems[0])
    for k in range(NFULL):
        pltpu.make_async_copy(rows[0],
                              acc_sh.at[pl.ds(s * RPW + k * CH, CH)],
                              ssems[0]).wait()
    pltpu.make_async_copy(rows[0].at[pl.ds(0, TAIL)],
                          acc_sh.at[pl.ds(s * RPW + NFULL * CH, TAIL)],
                          ssems[0]).wait()
    plsc.subcore_barrier()

    def g_wait(chunk, b):
        pltpu.make_async_copy(hp_hbm.at[src_v.at[chunk]], rows[b],
                              gsems[b]).wait()

    def s_issue(chunk, b):
        pltpu.async_copy(rows[b], acc_sh.at[dst_v.at[chunk]], ssems[b],
                         add=True)

    def s_wait(chunk, b):
        pltpu.make_async_copy(rows[b], acc_sh.at[dst_v.at[chunk]],
                              ssems[b]).wait()

    pltpu.async_copy(hp_hbm.at[src_v.at[0]], rows[0], gsems[0])
    pltpu.async_copy(hp_hbm.at[src_v.at[1]], rows[1], gsems[1])

    @pl.loop(0, NCHUNK - 2, step=3)
    def _(j):
        for b in range(3):
            ch = j + b
            nb = (b + 2) % 3

            @pl.when(ch >= 1)
            def _():
                s_wait(ch - 1, nb)

            @pl.when(ch + 2 <= NCHUNK - 1)
            def _():
                pltpu.async_copy(hp_hbm.at[src_v.at[ch + 2]], rows[nb],
                                 gsems[nb])
            g_wait(ch, b)
            s_issue(ch, b)

    for t in (NCHUNK - 2, NCHUNK - 1):
        b = t % 3
        s_wait(t - 1, (t + 2) % 3)
        g_wait(t, b)
        s_issue(t, b)
    s_wait(NCHUNK - 1, (NCHUNK - 1) % 3)

    plsc.subcore_barrier()
    pltpu.async_copy(acc_sh.at[pl.ds(s * RPW, RPW)],
                     out_hbm.at[c, pl.ds(s * RPW, RPW)], ssems[0])
    pltpu.make_async_copy(acc_sh.at[pl.ds(s * RPW, RPW)],
                          out_hbm.at[c, pl.ds(s * RPW, RPW)], ssems[0]).wait()


_edge_call = pl.kernel(
    _edge_body,
    out_type=jax.ShapeDtypeStruct((NC, N, F), jnp.float32),
    mesh=_MESH,
    scratch_types=[
        pltpu.VMEM((NCHUNK, CH), jnp.int32),
        pltpu.VMEM((NCHUNK, CH), jnp.int32),
        tuple(pltpu.VMEM((CH, F), jnp.float32) for _ in range(3)),
        tuple(pltpu.SemaphoreType.DMA for _ in range(3)),
        tuple(pltpu.SemaphoreType.DMA for _ in range(3)),
        pltpu.VMEM_SHARED((N, F), jnp.float32),
    ],
    compiler_params=_SC_PARAMS,
)


def _dinv_of(degp_ref):
    deg = degp_ref[0, :, 0] + degp_ref[1, :, 0] + 1.0
    return lax.rsqrt(jnp.maximum(deg, 1.0))


def _ln_gelu(aggv, g_ref, be_ref):
    mu = jnp.mean(aggv, axis=-1, keepdims=True)
    var = jnp.mean((aggv - mu) ** 2, axis=-1, keepdims=True)
    t = (aggv - mu) * lax.rsqrt(var + 1e-5) * g_ref[...] + be_ref[...]
    return jax.nn.gelu(t)


def _mm1_body(x_ref, w_ref, degp_ref, o_ref):
    dinv = _dinv_of(degp_ref)
    h = jnp.dot(x_ref[...], w_ref[...], preferred_element_type=jnp.float32)
    o_ref[...] = h * dinv[:, None]


_mm1_call = pl.pallas_call(
    _mm1_body,
    grid=(NBLK,),
    in_specs=[
        pl.BlockSpec((BLK, F), lambda i: (i, 0)),
        pl.BlockSpec((F, F), lambda i: (0, 0)),
        pl.BlockSpec((NC, BLK, DEGW), lambda i: (0, i, 0)),
    ],
    out_specs=pl.BlockSpec((BLK, F), lambda i: (i, 0)),
    out_shape=jax.ShapeDtypeStruct((N, F), jnp.float32),
)


def _mid_body(sp_ref, hp_ref, degp_ref, b_ref, g_ref, be_ref, w_ref, o_ref):
    dinv = _dinv_of(degp_ref)
    agg = (sp_ref[0] + sp_ref[1] + hp_ref[...]) * dinv[:, None] + b_ref[...]
    t = _ln_gelu(agg, g_ref, be_ref)
    h2 = jnp.dot(t, w_ref[...], preferred_element_type=jnp.float32)
    o_ref[...] = h2 * dinv[:, None]


_mid_call = pl.pallas_call(
    _mid_body,
    grid=(NBLK,),
    in_specs=[
        pl.BlockSpec((NC, BLK, F), lambda i: (0, i, 0)),
        pl.BlockSpec((BLK, F), lambda i: (i, 0)),
        pl.BlockSpec((NC, BLK, DEGW), lambda i: (0, i, 0)),
        pl.BlockSpec((1, F), lambda i: (0, 0)),
        pl.BlockSpec((1, F), lambda i: (0, 0)),
        pl.BlockSpec((1, F), lambda i: (0, 0)),
        pl.BlockSpec((F, F), lambda i: (0, 0)),
    ],
    out_specs=pl.BlockSpec((BLK, F), lambda i: (i, 0)),
    out_shape=jax.ShapeDtypeStruct((N, F), jnp.float32),
)


def _fin_body(sp_ref, hp_ref, degp_ref, b_ref, g_ref, be_ref, batch_ref,
              fcw_ref, fcb_ref, o_ref, pooled_acc, cnt_acc):
    i = pl.program_id(0)

    @pl.when(i == 0)
    def _():
        pooled_acc[...] = jnp.zeros((G, F), jnp.float32)
        cnt_acc[...] = jnp.zeros((G, F), jnp.float32)

    dinv = _dinv_of(degp_ref)
    agg = (sp_ref[0] + sp_ref[1] + hp_ref[...]) * dinv[:, None] + b_ref[...]
    t = _ln_gelu(agg, g_ref, be_ref)
    bvec = batch_ref[0, 0, :]
    onehot = (lax.broadcasted_iota(jnp.int32, (G, BLK), 0)
              == bvec[None, :]).astype(jnp.float32)
    pooled_acc[...] += jnp.dot(onehot, t, preferred_element_type=jnp.float32)
    cnt = jnp.sum(onehot, axis=1, keepdims=True)
    cnt_acc[...] += jnp.broadcast_to(cnt, (G, F))

    @pl.when(i == NBLK - 1)
    def _():
        pooled = pooled_acc[...] / jnp.maximum(cnt_acc[...], 1.0)
        o_ref[...] = (jnp.dot(pooled, fcw_ref[...],
                              preferred_element_type=jnp.float32)
                      + fcb_ref[...])


_fin_call = pl.pallas_call(
    _fin_body,
    grid=(NBLK,),
    in_specs=[
        pl.BlockSpec((NC, BLK, F), lambda i: (0, i, 0)),
        pl.BlockSpec((BLK, F), lambda i: (i, 0)),
        pl.BlockSpec((NC, BLK, DEGW), lambda i: (0, i, 0)),
        pl.BlockSpec((1, F), lambda i: (0, 0)),
        pl.BlockSpec((1, F), lambda i: (0, 0)),
        pl.BlockSpec((1, F), lambda i: (0, 0)),
        pl.BlockSpec((1, 1, BLK), lambda i: (i, 0, 0)),
        pl.BlockSpec((F, C), lambda i: (0, 0)),
        pl.BlockSpec((1, C), lambda i: (0, 0)),
    ],
    out_specs=pl.BlockSpec((G, C), lambda i: (0, 0)),
    out_shape=jax.ShapeDtypeStruct((G, C), jnp.float32),
    scratch_shapes=[
        pltpu.VMEM((G, F), jnp.float32),
        pltpu.VMEM((G, F), jnp.float32),
    ],
)


def kernel(x, edge_index, batch, W1, b1, g1, be1, W2, b2, g2, be2, fcW, fcb):
    src_r = edge_index[0].reshape(NW * NCHUNK, CH)
    dst_r = edge_index[1].reshape(NW * NCHUNK, CH)
    degp = _deg_call(dst_r)
    hp = _mm1_call(x, W1, degp)
    s1 = _edge_call(hp, src_r, dst_r)
    h2p = _mid_call(s1, hp, degp, b1.reshape(1, F), g1.reshape(1, F),
                    be1.reshape(1, F), W2)
    s2 = _edge_call(h2p, src_r, dst_r)
    out = _fin_call(s2, h2p, degp, b2.reshape(1, F), g2.reshape(1, F),
                    be2.reshape(1, F), batch.reshape(NBLK, 1, BLK), fcW,
                    fcb.reshape(1, C))
    return out


# speedup vs baseline: 1.0416x; 1.0416x over previous
import jax
import jax.numpy as jnp
from jax import lax
from jax.experimental import pallas as pl
from jax.experimental.pallas import tpu as pltpu
from jax.experimental.pallas import tpu_sc as plsc

N = 10000
E = 320000
F = 128
G = 8
C = 1000

NC, NS = 2, 16
NW = NC * NS
EPW = E // NW
CH = 80
FH = F // NC
EPT = E // NS
NCHT = EPT // CH
NCHUNK = EPW // CH
RPW = N // NS
NFULL = RPW // CH
TAIL = RPW - NFULL * CH
DEGW = 16

BLK = 1000
NBLK = N // BLK

_MESH = plsc.VectorSubcoreMesh(
    core_axis_name="c", subcore_axis_name="s", num_cores=NC, num_subcores=NS)

_SC_PARAMS = pltpu.CompilerParams(use_tc_tiling_on_sc=False)


def _deg_body(dst_hbm, out_hbm, dst_v, ones_v, z_v, deg_sh, dsem):
    c = lax.axis_index("c")
    s = lax.axis_index("s")
    w = s * NC + c

    pltpu.async_copy(dst_hbm.at[pl.ds(w * NCHUNK, NCHUNK)], dst_v, dsem)

    @pl.loop(0, CH)
    def _(i):
        ones_v[i] = jnp.ones((DEGW,), jnp.float32)

    @pl.loop(0, RPW)
    def _(i):
        z_v[i] = jnp.zeros((DEGW,), jnp.float32)

    pltpu.sync_copy(z_v, deg_sh.at[pl.ds(s * RPW, RPW)])
    pltpu.make_async_copy(dst_hbm.at[pl.ds(w * NCHUNK, NCHUNK)], dst_v,
                          dsem).wait()
    plsc.subcore_barrier()

    @pl.loop(0, NCHUNK)
    def _(j):
        pltpu.async_copy(ones_v, deg_sh.at[dst_v.at[j]], dsem, add=True)

    @pl.loop(0, NCHUNK)
    def _(j):
        pltpu.make_async_copy(ones_v, deg_sh.at[dst_v.at[j]], dsem).wait()

    plsc.subcore_barrier()
    pltpu.sync_copy(deg_sh.at[pl.ds(s * RPW, RPW)], z_v)
    pltpu.sync_copy(z_v, out_hbm.at[c, pl.ds(s * RPW, RPW)])


_deg_call = pl.kernel(
    _deg_body,
    out_type=jax.ShapeDtypeStruct((NC, N, DEGW), jnp.float32),
    mesh=_MESH,
    scratch_types=[
        pltpu.VMEM((NCHUNK, CH), jnp.int32),
        pltpu.VMEM((CH, DEGW), jnp.float32),
        pltpu.VMEM((RPW, DEGW), jnp.float32),
        pltpu.VMEM_SHARED((N, DEGW), jnp.float32),
        pltpu.SemaphoreType.DMA,
    ],
    compiler_params=_SC_PARAMS,
)


def _edge_body(hp_hbm, src_hbm, dst_hbm, out_hbm, src_v, dst_v, rows, gsems,
               ssems, acc_sh):
    c = lax.axis_index("c")
    s = lax.axis_index("s")
    w = s * NC + c

    pltpu.sync_copy(src_hbm.at[pl.ds(w * NCHUNK, NCHUNK)], src_v)
    pltpu.sync_copy(dst_hbm.at[pl.ds(w * NCHUNK, NCHUNK)], dst_v)

    @pl.loop(0, CH)
    def _(i):
        for jj in range(F // 16):
            rows[0][i, pl.ds(jj * 16, 16)] = jnp.zeros((16,), jnp.float32)

    for k in range(NFULL):
        pltpu.async_copy(rows[0], acc_sh.at[pl.ds(s * RPW + k * CH, CH)],
                         ssems[0])
    pltpu.async_copy(rows[0].at[pl.ds(0, TAIL)],
                     acc_sh.at[pl.ds(s * RPW + NFULL * CH, TAIL)], ssems[0])
    for k in range(NFULL):
        pltpu.make_async_copy(rows[0],
                              acc_sh.at[pl.ds(s * RPW + k * CH, CH)],
                              ssems[0]).wait()
    pltpu.make_async_copy(rows[0].at[pl.ds(0, TAIL)],
                          acc_sh.at[pl.ds(s * RPW + NFULL * CH, TAIL)],
                          ssems[0]).wait()
    plsc.subcore_barrier()

    def g_wait(chunk, b):
        pltpu.make_async_copy(hp_hbm.at[src_v.at[chunk]], rows[b],
                              gsems[b]).wait()

    def s_issue(chunk, b):
        pltpu.async_copy(rows[b], acc_sh.at[dst_v.at[chunk]], ssems[b],
                         add=True)

    def s_wait(chunk, b):
        pltpu.make_async_copy(rows[b], acc_sh.at[dst_v.at[chunk]],
                              ssems[b]).wait()

    pltpu.async_copy(hp_hbm.at[src_v.at[0]], rows[0], gsems[0])
    pltpu.async_copy(hp_hbm.at[src_v.at[1]], rows[1], gsems[1])

    @pl.loop(0, NCHUNK - 2, step=3)
    def _(j):
        for b in range(3):
            ch = j + b
            nb = (b + 2) % 3

            @pl.when(ch >= 1)
            def _():
                s_wait(ch - 1, nb)

            @pl.when(ch + 2 <= NCHUNK - 1)
            def _():
                pltpu.async_copy(hp_hbm.at[src_v.at[ch + 2]], rows[nb],
                                 gsems[nb])
            g_wait(ch, b)
            s_issue(ch, b)

    for t in (NCHUNK - 2, NCHUNK - 1):
        b = t % 3
        s_wait(t - 1, (t + 2) % 3)
        g_wait(t, b)
        s_issue(t, b)
    s_wait(NCHUNK - 1, (NCHUNK - 1) % 3)

    plsc.subcore_barrier()
    nw_slices = [(k * CH, CH) for k in range(NFULL)] + [(NFULL * CH, TAIL)]
    for k, (off, ln) in enumerate(nw_slices):
        r = k % 3
        if k >= 3:
            po, pln = nw_slices[k - 3]
            pltpu.make_async_copy(
                rows[r].at[pl.ds(0, pln)],
                out_hbm.at[c, pl.ds(s * RPW + po, pln)], ssems[r]).wait()
        pltpu.async_copy(acc_sh.at[pl.ds(s * RPW + off, ln)],
                         rows[r].at[pl.ds(0, ln)], gsems[r])
        pltpu.make_async_copy(acc_sh.at[pl.ds(s * RPW + off, ln)],
                              rows[r].at[pl.ds(0, ln)], gsems[r]).wait()
        pltpu.async_copy(rows[r].at[pl.ds(0, ln)],
                         out_hbm.at[c, pl.ds(s * RPW + off, ln)], ssems[r])
    for k in range(len(nw_slices) - 3, len(nw_slices)):
        off, ln = nw_slices[k]
        pltpu.make_async_copy(rows[k % 3].at[pl.ds(0, ln)],
                              out_hbm.at[c, pl.ds(s * RPW + off, ln)],
                              ssems[k % 3]).wait()


_edge_call = pl.kernel(
    _edge_body,
    out_type=jax.ShapeDtypeStruct((NC, N, F), jnp.float32),
    mesh=_MESH,
    scratch_types=[
        pltpu.VMEM((NCHUNK, CH), jnp.int32),
        pltpu.VMEM((NCHUNK, CH), jnp.int32),
        tuple(pltpu.VMEM((CH, F), jnp.float32) for _ in range(3)),
        tuple(pltpu.SemaphoreType.DMA for _ in range(3)),
        tuple(pltpu.SemaphoreType.DMA for _ in range(3)),
        pltpu.VMEM_SHARED((N, F), jnp.float32),
    ],
    compiler_params=_SC_PARAMS,
)


def _dinv_of(degp_ref):
    deg = degp_ref[0, :, 0] + degp_ref[1, :, 0] + 1.0
    return lax.rsqrt(jnp.maximum(deg, 1.0))


def _ln_gelu(aggv, g_ref, be_ref):
    mu = jnp.mean(aggv, axis=-1, keepdims=True)
    var = jnp.mean((aggv - mu) ** 2, axis=-1, keepdims=True)
    t = (aggv - mu) * lax.rsqrt(var + 1e-5) * g_ref[...] + be_ref[...]
    return jax.nn.gelu(t)


def _mm1_body(x_ref, w_ref, degp_ref, o_ref):
    dinv = _dinv_of(degp_ref)
    h = jnp.dot(x_ref[...], w_ref[...], preferred_element_type=jnp.float32)
    o_ref[...] = h * dinv[:, None]


_mm1_call = pl.pallas_call(
    _mm1_body,
    grid=(NBLK,),
    in_specs=[
        pl.BlockSpec((BLK, F), lambda i: (i, 0)),
        pl.BlockSpec((F, F), lambda i: (0, 0)),
        pl.BlockSpec((NC, BLK, DEGW), lambda i: (0, i, 0)),
    ],
    out_specs=pl.BlockSpec((BLK, F), lambda i: (i, 0)),
    out_shape=jax.ShapeDtypeStruct((N, F), jnp.float32),
)


def _mid_body(sp_ref, hp_ref, degp_ref, b_ref, g_ref, be_ref, w_ref, o_ref):
    dinv = _dinv_of(degp_ref)
    agg = (sp_ref[0] + sp_ref[1] + hp_ref[...]) * dinv[:, None] + b_ref[...]
    t = _ln_gelu(agg, g_ref, be_ref)
    h2 = jnp.dot(t, w_ref[...], preferred_element_type=jnp.float32)
    o_ref[...] = h2 * dinv[:, None]


_mid_call = pl.pallas_call(
    _mid_body,
    grid=(NBLK,),
    in_specs=[
        pl.BlockSpec((NC, BLK, F), lambda i: (0, i, 0)),
        pl.BlockSpec((BLK, F), lambda i: (i, 0)),
        pl.BlockSpec((NC, BLK, DEGW), lambda i: (0, i, 0)),
        pl.BlockSpec((1, F), lambda i: (0, 0)),
        pl.BlockSpec((1, F), lambda i: (0, 0)),
        pl.BlockSpec((1, F), lambda i: (0, 0)),
        pl.BlockSpec((F, F), lambda i: (0, 0)),
    ],
    out_specs=pl.BlockSpec((BLK, F), lambda i: (i, 0)),
    out_shape=jax.ShapeDtypeStruct((N, F), jnp.float32),
)


def _fin_body(sp_ref, hp_ref, degp_ref, b_ref, g_ref, be_ref, batch_ref,
              fcw_ref, fcb_ref, o_ref, pooled_acc, cnt_acc):
    i = pl.program_id(0)

    @pl.when(i == 0)
    def _():
        pooled_acc[...] = jnp.zeros((G, F), jnp.float32)
        cnt_acc[...] = jnp.zeros((G, F), jnp.float32)

    dinv = _dinv_of(degp_ref)
    agg = (sp_ref[0] + sp_ref[1] + hp_ref[...]) * dinv[:, None] + b_ref[...]
    t = _ln_gelu(agg, g_ref, be_ref)
    bvec = batch_ref[0, 0, :]
    onehot = (lax.broadcasted_iota(jnp.int32, (G, BLK), 0)
              == bvec[None, :]).astype(jnp.float32)
    pooled_acc[...] += jnp.dot(onehot, t, preferred_element_type=jnp.float32)
    cnt = jnp.sum(onehot, axis=1, keepdims=True)
    cnt_acc[...] += jnp.broadcast_to(cnt, (G, F))

    @pl.when(i == NBLK - 1)
    def _():
        pooled = pooled_acc[...] / jnp.maximum(cnt_acc[...], 1.0)
        o_ref[...] = (jnp.dot(pooled, fcw_ref[...],
                              preferred_element_type=jnp.float32)
                      + fcb_ref[...])


_fin_call = pl.pallas_call(
    _fin_body,
    grid=(NBLK,),
    in_specs=[
        pl.BlockSpec((NC, BLK, F), lambda i: (0, i, 0)),
        pl.BlockSpec((BLK, F), lambda i: (i, 0)),
        pl.BlockSpec((NC, BLK, DEGW), lambda i: (0, i, 0)),
        pl.BlockSpec((1, F), lambda i: (0, 0)),
        pl.BlockSpec((1, F), lambda i: (0, 0)),
        pl.BlockSpec((1, F), lambda i: (0, 0)),
        pl.BlockSpec((1, 1, BLK), lambda i: (i, 0, 0)),
        pl.BlockSpec((F, C), lambda i: (0, 0)),
        pl.BlockSpec((1, C), lambda i: (0, 0)),
    ],
    out_specs=pl.BlockSpec((G, C), lambda i: (0, 0)),
    out_shape=jax.ShapeDtypeStruct((G, C), jnp.float32),
    scratch_shapes=[
        pltpu.VMEM((G, F), jnp.float32),
        pltpu.VMEM((G, F), jnp.float32),
    ],
)


def kernel(x, edge_index, batch, W1, b1, g1, be1, W2, b2, g2, be2, fcW, fcb):
    src_r = edge_index[0].reshape(NW * NCHUNK, CH)
    dst_r = edge_index[1].reshape(NW * NCHUNK, CH)
    degp = _deg_call(dst_r)
    hp = _mm1_call(x, W1, degp)
    s1 = _edge_call(hp, src_r, dst_r)
    h2p = _mid_call(s1, hp, degp, b1.reshape(1, F), g1.reshape(1, F),
                    be1.reshape(1, F), W2)
    s2 = _edge_call(h2p, src_r, dst_r)
    out = _fin_call(s2, h2p, degp, b2.reshape(1, F), g2.reshape(1, F),
                    be2.reshape(1, F), batch.reshape(NBLK, 1, BLK), fcW,
                    fcb.reshape(1, C))
    return out
